# baseline (speedup 1.0000x reference)
import functools

import jax
import jax.numpy as jnp
from jax.experimental import pallas as pl

T = 16384
D = 4096
E = 64
TILE_T = 512


def _router_kernel(x_ref, w_ref, out_w_ref, out_idx_ref):
    logits = jax.lax.dot_general(
        x_ref[...], w_ref[...],
        dimension_numbers=(((1,), (1,)), ((), ())),
        preferred_element_type=jnp.float32,
    )
    m = jnp.max(logits, axis=-1)
    idx = jnp.argmax(logits, axis=-1).astype(jnp.int32)
    s = jnp.sum(jnp.exp(logits - m[:, None]), axis=-1)
    out_w_ref[...] = 1.0 / s
    out_idx_ref[...] = idx


@functools.partial(jax.jit, static_argnames=())
def kernel(x, W):
    grid = (T // TILE_T,)
    w, idx = pl.pallas_call(
        _router_kernel,
        grid=grid,
        in_specs=[
            pl.BlockSpec((TILE_T, D), lambda i: (i, 0)),
            pl.BlockSpec((E, D), lambda i: (0, 0)),
        ],
        out_specs=[
            pl.BlockSpec((TILE_T,), lambda i: (i,)),
            pl.BlockSpec((TILE_T,), lambda i: (i,)),
        ],
        out_shape=[
            jax.ShapeDtypeStruct((T,), jnp.float32),
            jax.ShapeDtypeStruct((T,), jnp.int32),
        ],
    )(x, W)
    return (w, idx)


# speedup vs baseline: 1.0911x; 1.0911x over previous
import functools

import jax
import jax.numpy as jnp
from jax.experimental import pallas as pl

T = 16384
D = 4096
E = 64
TILE_T = 1024


def _router_kernel(x_ref, w_ref, out_w_ref, out_idx_ref):
    logits = jax.lax.dot_general(
        x_ref[...], w_ref[...],
        dimension_numbers=(((1,), (1,)), ((), ())),
        preferred_element_type=jnp.float32,
    )
    m = jnp.max(logits, axis=-1)
    idx = jnp.argmax(logits, axis=-1).astype(jnp.int32)
    s = jnp.sum(jnp.exp(logits - m[:, None]), axis=-1)
    out_w_ref[...] = 1.0 / s
    out_idx_ref[...] = idx


@functools.partial(jax.jit, static_argnames=())
def kernel(x, W):
    grid = (T // TILE_T,)
    w, idx = pl.pallas_call(
        _router_kernel,
        grid=grid,
        in_specs=[
            pl.BlockSpec((TILE_T, D), lambda i: (i, 0)),
            pl.BlockSpec((E, D), lambda i: (0, 0)),
        ],
        out_specs=[
            pl.BlockSpec((TILE_T,), lambda i: (i,)),
            pl.BlockSpec((TILE_T,), lambda i: (i,)),
        ],
        out_shape=[
            jax.ShapeDtypeStruct((T,), jnp.float32),
            jax.ShapeDtypeStruct((T,), jnp.int32),
        ],
    )(x, W)
    return (w, idx)


# speedup vs baseline: 1.0940x; 1.0026x over previous
import functools

import jax
import jax.numpy as jnp
from jax.experimental import pallas as pl
from jax.experimental.pallas import tpu as pltpu

T = 16384
D = 4096
E = 64
TILE_T = 1024


def _router_kernel(x_ref, w_ref, out_w_ref, out_idx_ref):
    logits = jax.lax.dot_general(
        x_ref[...], w_ref[...],
        dimension_numbers=(((1,), (1,)), ((), ())),
        preferred_element_type=jnp.float32,
    )
    m = jnp.max(logits, axis=-1)
    idx = jnp.argmax(logits, axis=-1).astype(jnp.int32)
    s = jnp.sum(jnp.exp(logits - m[:, None]), axis=-1)
    out_w_ref[...] = 1.0 / s
    out_idx_ref[...] = idx


@functools.partial(jax.jit, static_argnames=())
def kernel(x, W):
    grid = (T // TILE_T,)
    w, idx = pl.pallas_call(
        _router_kernel,
        grid=grid,
        in_specs=[
            pl.BlockSpec((TILE_T, D), lambda i: (i, 0)),
            pl.BlockSpec((E, D), lambda i: (0, 0)),
        ],
        out_specs=[
            pl.BlockSpec((TILE_T,), lambda i: (i,)),
            pl.BlockSpec((TILE_T,), lambda i: (i,)),
        ],
        out_shape=[
            jax.ShapeDtypeStruct((T,), jnp.float32),
            jax.ShapeDtypeStruct((T,), jnp.int32),
        ],
        compiler_params=pltpu.CompilerParams(
            dimension_semantics=("parallel",),
            vmem_limit_bytes=128 * 1024 * 1024,
        ),
    )(x, W)
    return (w, idx)
